# baseline (speedup 1.0000x reference)
import math

import jax
import jax.numpy as jnp
from jax.experimental import pallas as pl
from jax.experimental.pallas import tpu as pltpu

_K0 = math.sqrt(2.0 / math.pi)
_K1 = 0.044715 * _K0

_CHUNK = 1024
_NIN = 4
_NOUT = 3


def _gelu(x):
    x2 = x * x
    z = x * (_K1 * x2 + _K0)
    hx = 0.5 * x
    return hx + hx * jnp.tanh(z)


def _pipeline_body(x_hbm, o_hbm, in_buf, out_buf, in_sem, out_sem):
    rows = x_hbm.shape[0]
    nchunk = rows // _CHUNK

    def in_copy(i, slot):
        return pltpu.make_async_copy(
            x_hbm.at[pl.ds(i * _CHUNK, _CHUNK), :], in_buf.at[slot],
            in_sem.at[slot])

    def out_copy(i, slot):
        return pltpu.make_async_copy(
            out_buf.at[slot], o_hbm.at[pl.ds(i * _CHUNK, _CHUNK), :],
            out_sem.at[slot])

    for j in range(min(_NIN, nchunk)):
        in_copy(j, j).start()

    for i in range(nchunk):
        islot = i % _NIN
        oslot = i % _NOUT
        in_copy(i, islot).wait()
        if i >= _NOUT:
            out_copy(i - _NOUT, oslot).wait()
        out_buf[oslot] = _gelu(in_buf[islot])
        out_copy(i, oslot).start()
        if i + _NIN < nchunk:
            in_copy(i + _NIN, islot).start()

    for j in range(max(nchunk - _NOUT, 0), nchunk):
        out_copy(j, j % _NOUT).wait()


def kernel(x, log_tau, log_blend):
    B, T, D = x.shape
    rows = B * T
    x2 = x.reshape(rows, D)
    out = pl.pallas_call(
        _pipeline_body,
        in_specs=[pl.BlockSpec(memory_space=pltpu.MemorySpace.HBM)],
        out_specs=pl.BlockSpec(memory_space=pltpu.MemorySpace.HBM),
        out_shape=jax.ShapeDtypeStruct((rows, D), x.dtype),
        scratch_shapes=[
            pltpu.VMEM((_NIN, _CHUNK, D), jnp.float32),
            pltpu.VMEM((_NOUT, _CHUNK, D), jnp.float32),
            pltpu.SemaphoreType.DMA((_NIN,)),
            pltpu.SemaphoreType.DMA((_NOUT,)),
        ],
        compiler_params=pltpu.CompilerParams(
            vmem_limit_bytes=100 * 1024 * 1024,
        ),
    )(x2)
    return out.reshape(B, T, D)
